# baseline (speedup 1.0000x reference)
import functools

import jax
import jax.numpy as jnp
from jax import lax
from jax.experimental import pallas as pl
from jax.experimental.pallas import tpu as pltpu
from jax.experimental.pallas import tpu_sc as plsc

B, T_TEXT, T_MEL, N_MELS = 16, 512, 2048, 80
LANES = 16

_INFO = plsc.get_sparse_core_info()
NC = _INFO.num_cores
NS = _INFO.num_subcores
NW = NC * NS

SC_B = 8
TC_B = B - SC_B
TPB = NW // SC_B

BLK = 8 * 128
BLOCKS_PER_B = T_MEL * N_MELS // BLK
BLOCKS_PER_TILE = BLOCKS_PER_B // TPB
CHUNK_BLOCKS = 5
N_CHUNKS = BLOCKS_PER_TILE // CHUNK_BLOCKS
CHUNK_W = CHUNK_BLOCKS * BLK

OUT_W = 4 * LANES


def _sc_body(mt, mp, mq, vm, out,
             bt, bp, bq, validbuf, outbuf, sem0, sem1, sem2):
    wid = lax.axis_index("s") * NC + lax.axis_index("c")
    beta0 = wid * BLOCKS_PER_TILE
    mel_base = beta0 * BLK
    b_idx = wid // TPB

    sems = (sem0, sem1)

    def start_chunk(c, slot):
        base = mel_base + c * CHUNK_W
        boff = slot * CHUNK_W
        sem = sems[slot]
        pltpu.async_copy(mt.at[pl.ds(base, CHUNK_W)],
                         bt.at[pl.ds(boff, CHUNK_W)], sem)
        pltpu.async_copy(mp.at[pl.ds(base, CHUNK_W)],
                         bp.at[pl.ds(boff, CHUNK_W)], sem)
        pltpu.async_copy(mq.at[pl.ds(base, CHUNK_W)],
                         bq.at[pl.ds(boff, CHUNK_W)], sem)

    def wait_chunk(slot):
        for dst in (bt, bp, bq):
            pltpu.make_async_copy(mt.at[pl.ds(0, CHUNK_W)],
                                  dst.at[pl.ds(0, CHUNK_W)],
                                  sems[slot]).wait()

    start_chunk(0, 0)
    start_chunk(1, 1)

    b1 = b_idx // 8
    b0 = lax.rem(b_idx, 8)
    vrow0 = (b1 * (T_MEL // 128) * 8 + b0) * 128

    def fire_row(t1, _):
        pltpu.async_copy(vm.at[pl.ds(vrow0 + t1 * 1024, 128)],
                         validbuf.at[pl.ds(t1 * 128, 128)], sem2)
        return 0

    lax.fori_loop(0, T_MEL // 128, fire_row, 0)

    def drain_row(t1, _):
        pltpu.make_async_copy(vm.at[pl.ds(0, 128)],
                              validbuf.at[pl.ds(0, 128)], sem2).wait()
        return 0

    lax.fori_loop(0, T_MEL // 128, drain_row, 0)

    zv = jnp.zeros((LANES,), jnp.float32)

    def chunk_sums(c, boff, carry):
        cbeta = beta0 + c * CHUNK_BLOCKS

        def blk_body(r, rc):
            t1 = lax.rem(cbeta + r, T_MEL // 128)
            vbase = t1 * 128
            dbase = boff + r * BLK

            def one_group(jo, jc):
                accp, accq = jc
                wvec = validbuf[pl.ds(vbase + jo, LANES)]
                sp = None
                sq = None
                for m0 in range(8):
                    o = dbase + m0 * 128 + jo
                    t = bt[pl.ds(o, LANES)]
                    p = bp[pl.ds(o, LANES)]
                    q = bq[pl.ds(o, LANES)]
                    a = jnp.abs(p - t)
                    b = jnp.abs(q - t)
                    sp = a if sp is None else sp + a
                    sq = b if sq is None else sq + b
                return accp + wvec * sp, accq + wvec * sq

            for u in range(8):
                rc = one_group(u * LANES, rc)
            return rc

        return lax.fori_loop(0, CHUNK_BLOCKS, blk_body, carry)

    def chunk_body(c, carry):
        par = lax.rem(c, 2)

        @pl.when(par == 0)
        def _():
            wait_chunk(0)

        @pl.when(par == 1)
        def _():
            wait_chunk(1)

        carry = chunk_sums(c, par * CHUNK_W, carry)

        @pl.when(c < N_CHUNKS - 2)
        def _():
            @pl.when(par == 0)
            def _():
                start_chunk(c + 2, 0)

            @pl.when(par == 1)
            def _():
                start_chunk(c + 2, 1)

        return carry

    accp, accq = lax.fori_loop(0, N_CHUNKS, chunk_body, (zv, zv))

    vals = (accp, accq, zv, zv)
    for k, v in enumerate(vals):
        outbuf[pl.ds(k * LANES, LANES)] = v
    pltpu.sync_copy(outbuf, out.at[pl.ds(wid * OUT_W, OUT_W)])


_sc_partials = functools.partial(
    pl.kernel,
    mesh=plsc.VectorSubcoreMesh(core_axis_name="c", subcore_axis_name="s"),
    out_type=jax.ShapeDtypeStruct((NW * OUT_W,), jnp.float32),
    scratch_types=[
        pltpu.VMEM((2 * CHUNK_W,), jnp.float32),
        pltpu.VMEM((2 * CHUNK_W,), jnp.float32),
        pltpu.VMEM((2 * CHUNK_W,), jnp.float32),
        pltpu.VMEM((T_MEL,), jnp.float32),
        pltpu.VMEM((OUT_W,), jnp.float32),
        pltpu.SemaphoreType.DMA,
        pltpu.SemaphoreType.DMA,
        pltpu.SemaphoreType.DMA,
    ],
)(_sc_body)


def _tc_mel_body(tref, pref, qref, wref, outp_ref, outq_ref):
    t = tref[0]
    p = pref[0]
    q = qref[0]
    w = jnp.where(wref[0], 0.0, 1.0)
    sp = jnp.sum(jnp.abs(p - t), axis=0, keepdims=True)
    sq = jnp.sum(jnp.abs(q - t), axis=0, keepdims=True)
    outp_ref[0] = sp * w
    outq_ref[0] = sq * w


def _tc_mel(mel_t, mel_p, mel_q, mask3d):
    mel_spec = pl.BlockSpec((1, N_MELS, T_MEL), lambda g: (g + SC_B, 0, 0))
    w_spec = pl.BlockSpec((1, 1, T_MEL), lambda g: (g + SC_B, 0, 0))
    row = jax.ShapeDtypeStruct((TC_B, 1, T_MEL), jnp.float32)
    outs = pl.pallas_call(
        _tc_mel_body,
        grid=(TC_B,),
        in_specs=[mel_spec, mel_spec, mel_spec, w_spec],
        out_specs=(pl.BlockSpec((1, 1, T_MEL), lambda g: (g, 0, 0)),) * 2,
        out_shape=(row, row),
    )(mel_t, mel_p, mel_q, mask3d)
    return outs


def _reduce_body(pref, tcp_ref, tcq_ref, dt_ref, dp_ref, pt_ref, pp_ref,
                 et_ref, ep_ref, vmask_ref, mmask_ref,
                 dur_ref, pit_ref, ene_ref, mel_ref, post_ref):
    x = pref[...]
    s = jnp.sum(x, axis=0)

    def seg(k):
        return (jnp.sum(s[k * LANES:(k + 1) * LANES])
                + jnp.sum(s[OUT_W + k * LANES:OUT_W + (k + 1) * LANES]))

    w = jnp.where(vmask_ref[...], 0.0, 1.0)
    cnt_var = jnp.sum(w)
    d0 = dp_ref[...] - dt_ref[...]
    d1 = pp_ref[...] - pt_ref[...]
    d2 = ep_ref[...] - et_ref[...]
    dur_ref[0, 0] = jnp.sum(w * d0 * d0) / cnt_var
    pit_ref[0, 0] = jnp.sum(w * d1 * d1) / cnt_var
    ene_ref[0, 0] = jnp.sum(w * d2 * d2) / cnt_var

    wm = jnp.where(mmask_ref[...], 0.0, 1.0)
    cnt_mel = jnp.sum(wm) * jnp.float32(N_MELS)
    mel_ref[0, 0] = (seg(0) + jnp.sum(tcp_ref[...])) / cnt_mel
    post_ref[0, 0] = (seg(1) + jnp.sum(tcq_ref[...])) / cnt_mel


def _mel_flat(x):
    y = x.transpose(0, 2, 1)
    y = y.reshape(B, N_MELS // 8, 8, T_MEL // 128, 128)
    y = y.transpose(0, 1, 3, 2, 4)
    return y.reshape(-1)


def _rowmajor_flat(x, rows, cols):
    y = x.reshape(rows // 8, 8, cols // 128, 128)
    return y.transpose(0, 2, 1, 3).reshape(-1)


def kernel(duration_target, duration_prediction, pitch_target,
           pitch_prediction, energy_target, energy_prediction,
           variance_mask, mel_target, mel_prediction,
           postnet_mel_prediction, mel_mask):
    mt = _mel_flat(mel_target)
    mp = _mel_flat(mel_prediction)
    mq = _mel_flat(postnet_mel_prediction)
    vm = _rowmajor_flat(
        jnp.logical_not(mel_mask).astype(jnp.float32).reshape(B, T_MEL),
        B, T_MEL)

    partials = _sc_partials(mt, mp, mq, vm)

    mask3d = mel_mask.reshape(B, 1, T_MEL)
    tcp, tcq = _tc_mel(mel_target.transpose(0, 2, 1),
                       mel_prediction.transpose(0, 2, 1),
                       postnet_mel_prediction.transpose(0, 2, 1), mask3d)

    scalar = jax.ShapeDtypeStruct((1, 1), jnp.float32)
    outs = pl.pallas_call(
        _reduce_body,
        out_shape=(scalar,) * 5,
        out_specs=(pl.BlockSpec(memory_space=pltpu.SMEM),) * 5,
    )(partials.reshape(NW // 2, 2 * OUT_W), tcp, tcq,
      duration_target, duration_prediction, pitch_target, pitch_prediction,
      energy_target, energy_prediction, variance_mask, mask3d)
    return tuple(o.reshape(()) for o in outs)


# speedup vs baseline: 1.7512x; 1.7512x over previous
import functools

import jax
import jax.numpy as jnp
from jax import lax
from jax.experimental import pallas as pl
from jax.experimental.pallas import tpu as pltpu
from jax.experimental.pallas import tpu_sc as plsc

B, T_TEXT, T_MEL, N_MELS = 16, 512, 2048, 80
LANES = 16

_INFO = plsc.get_sparse_core_info()
NC = _INFO.num_cores
NS = _INFO.num_subcores
NW = NC * NS

SC_B = 8
TC_B = B - SC_B
TPB = NW // SC_B

BLK = 8 * 128
BLOCKS_PER_B = T_MEL * N_MELS // BLK
BLOCKS_PER_TILE = BLOCKS_PER_B // TPB
CHUNK_BLOCKS = 5
N_CHUNKS = BLOCKS_PER_TILE // CHUNK_BLOCKS
CHUNK_W = CHUNK_BLOCKS * BLK

OUT_W = 4 * LANES


def _sc_body(mt, mp, mq, vm, out,
             bt, bp, bq, validbuf, outbuf, sem0, sem1, sem2):
    wid = lax.axis_index("s") * NC + lax.axis_index("c")
    beta0 = wid * BLOCKS_PER_TILE
    mel_base = beta0 * BLK
    b_idx = wid // TPB

    sems = (sem0, sem1)

    def start_chunk(c, slot):
        base = mel_base + c * CHUNK_W
        boff = slot * CHUNK_W
        sem = sems[slot]
        pltpu.async_copy(mt.at[pl.ds(base, CHUNK_W)],
                         bt.at[pl.ds(boff, CHUNK_W)], sem)
        pltpu.async_copy(mp.at[pl.ds(base, CHUNK_W)],
                         bp.at[pl.ds(boff, CHUNK_W)], sem)
        pltpu.async_copy(mq.at[pl.ds(base, CHUNK_W)],
                         bq.at[pl.ds(boff, CHUNK_W)], sem)

    def wait_chunk(slot):
        for dst in (bt, bp, bq):
            pltpu.make_async_copy(mt.at[pl.ds(0, CHUNK_W)],
                                  dst.at[pl.ds(0, CHUNK_W)],
                                  sems[slot]).wait()

    start_chunk(0, 0)
    start_chunk(1, 1)

    b1 = b_idx // 8
    b0 = lax.rem(b_idx, 8)
    vrow0 = (b1 * (T_MEL // 128) * 8 + b0) * 128

    def fire_row(t1, _):
        pltpu.async_copy(vm.at[pl.ds(vrow0 + t1 * 1024, 128)],
                         validbuf.at[pl.ds(t1 * 128, 128)], sem2)
        return 0

    lax.fori_loop(0, T_MEL // 128, fire_row, 0)

    def drain_row(t1, _):
        pltpu.make_async_copy(vm.at[pl.ds(0, 128)],
                              validbuf.at[pl.ds(0, 128)], sem2).wait()
        return 0

    lax.fori_loop(0, T_MEL // 128, drain_row, 0)

    zv = jnp.zeros((LANES,), jnp.float32)

    def chunk_sums(c, boff, carry):
        cbeta = beta0 + c * CHUNK_BLOCKS

        def blk_body(r, rc):
            t1 = lax.rem(cbeta + r, T_MEL // 128)
            vbase = t1 * 128
            dbase = boff + r * BLK

            def one_group(jo, jc):
                accp, accq = jc
                wvec = validbuf[pl.ds(vbase + jo, LANES)]
                sp = None
                sq = None
                for m0 in range(8):
                    o = dbase + m0 * 128 + jo
                    t = bt[pl.ds(o, LANES)]
                    p = bp[pl.ds(o, LANES)]
                    q = bq[pl.ds(o, LANES)]
                    a = jnp.abs(p - t)
                    b = jnp.abs(q - t)
                    sp = a if sp is None else sp + a
                    sq = b if sq is None else sq + b
                return accp + wvec * sp, accq + wvec * sq

            def j_body(j, jc):
                jo = j * (4 * LANES)
                for u in range(4):
                    jc = one_group(jo + u * LANES, jc)
                return jc

            return lax.fori_loop(0, 2, j_body, rc)

        return lax.fori_loop(0, CHUNK_BLOCKS, blk_body, carry)

    def chunk_body(c, carry):
        par = lax.rem(c, 2)

        @pl.when(par == 0)
        def _():
            wait_chunk(0)

        @pl.when(par == 1)
        def _():
            wait_chunk(1)

        carry = chunk_sums(c, par * CHUNK_W, carry)

        @pl.when(c < N_CHUNKS - 2)
        def _():
            @pl.when(par == 0)
            def _():
                start_chunk(c + 2, 0)

            @pl.when(par == 1)
            def _():
                start_chunk(c + 2, 1)

        return carry

    accp, accq = lax.fori_loop(0, N_CHUNKS, chunk_body, (zv, zv))

    vals = (accp, accq, zv, zv)
    for k, v in enumerate(vals):
        outbuf[pl.ds(k * LANES, LANES)] = v
    pltpu.sync_copy(outbuf, out.at[pl.ds(wid * OUT_W, OUT_W)])


_sc_partials = functools.partial(
    pl.kernel,
    mesh=plsc.VectorSubcoreMesh(core_axis_name="c", subcore_axis_name="s"),
    out_type=jax.ShapeDtypeStruct((NW * OUT_W,), jnp.float32),
    scratch_types=[
        pltpu.VMEM((2 * CHUNK_W,), jnp.float32),
        pltpu.VMEM((2 * CHUNK_W,), jnp.float32),
        pltpu.VMEM((2 * CHUNK_W,), jnp.float32),
        pltpu.VMEM((T_MEL,), jnp.float32),
        pltpu.VMEM((OUT_W,), jnp.float32),
        pltpu.SemaphoreType.DMA,
        pltpu.SemaphoreType.DMA,
        pltpu.SemaphoreType.DMA,
    ],
)(_sc_body)


def _tc_mel_body(tref, pref, qref, wref, outp_ref, outq_ref):
    t = tref[0]
    p = pref[0]
    q = qref[0]
    w = jnp.where(wref[0], 0.0, 1.0)
    sp = jnp.sum(jnp.abs(p - t), axis=0, keepdims=True)
    sq = jnp.sum(jnp.abs(q - t), axis=0, keepdims=True)
    outp_ref[0] = sp * w
    outq_ref[0] = sq * w


def _tc_mel(mel_t, mel_p, mel_q, mask3d):
    mel_spec = pl.BlockSpec((1, N_MELS, T_MEL), lambda g: (g + SC_B, 0, 0))
    w_spec = pl.BlockSpec((1, 1, T_MEL), lambda g: (g + SC_B, 0, 0))
    row = jax.ShapeDtypeStruct((TC_B, 1, T_MEL), jnp.float32)
    outs = pl.pallas_call(
        _tc_mel_body,
        grid=(TC_B,),
        in_specs=[mel_spec, mel_spec, mel_spec, w_spec],
        out_specs=(pl.BlockSpec((1, 1, T_MEL), lambda g: (g, 0, 0)),) * 2,
        out_shape=(row, row),
    )(mel_t, mel_p, mel_q, mask3d)
    return outs


def _reduce_body(pref, tcp_ref, tcq_ref, dt_ref, dp_ref, pt_ref, pp_ref,
                 et_ref, ep_ref, vmask_ref, mmask_ref,
                 dur_ref, pit_ref, ene_ref, mel_ref, post_ref):
    x = pref[...]
    s = jnp.sum(x, axis=0)

    def seg(k):
        return (jnp.sum(s[k * LANES:(k + 1) * LANES])
                + jnp.sum(s[OUT_W + k * LANES:OUT_W + (k + 1) * LANES]))

    w = jnp.where(vmask_ref[...], 0.0, 1.0)
    cnt_var = jnp.sum(w)
    d0 = dp_ref[...] - dt_ref[...]
    d1 = pp_ref[...] - pt_ref[...]
    d2 = ep_ref[...] - et_ref[...]
    dur_ref[0, 0] = jnp.sum(w * d0 * d0) / cnt_var
    pit_ref[0, 0] = jnp.sum(w * d1 * d1) / cnt_var
    ene_ref[0, 0] = jnp.sum(w * d2 * d2) / cnt_var

    wm = jnp.where(mmask_ref[...], 0.0, 1.0)
    cnt_mel = jnp.sum(wm) * jnp.float32(N_MELS)
    mel_ref[0, 0] = (seg(0) + jnp.sum(tcp_ref[...])) / cnt_mel
    post_ref[0, 0] = (seg(1) + jnp.sum(tcq_ref[...])) / cnt_mel


def _mel_flat(x):
    y = x.transpose(0, 2, 1)
    y = y.reshape(B, N_MELS // 8, 8, T_MEL // 128, 128)
    y = y.transpose(0, 1, 3, 2, 4)
    return y.reshape(-1)


def _rowmajor_flat(x, rows, cols):
    y = x.reshape(rows // 8, 8, cols // 128, 128)
    return y.transpose(0, 2, 1, 3).reshape(-1)


def kernel(duration_target, duration_prediction, pitch_target,
           pitch_prediction, energy_target, energy_prediction,
           variance_mask, mel_target, mel_prediction,
           postnet_mel_prediction, mel_mask):
    mt = _mel_flat(mel_target)
    mp = _mel_flat(mel_prediction)
    mq = _mel_flat(postnet_mel_prediction)
    vm = _rowmajor_flat(
        jnp.logical_not(mel_mask).astype(jnp.float32).reshape(B, T_MEL),
        B, T_MEL)

    partials = _sc_partials(mt, mp, mq, vm)

    mask3d = mel_mask.reshape(B, 1, T_MEL)
    tcp, tcq = _tc_mel(mel_target.transpose(0, 2, 1),
                       mel_prediction.transpose(0, 2, 1),
                       postnet_mel_prediction.transpose(0, 2, 1), mask3d)

    scalar = jax.ShapeDtypeStruct((1, 1), jnp.float32)
    outs = pl.pallas_call(
        _reduce_body,
        out_shape=(scalar,) * 5,
        out_specs=(pl.BlockSpec(memory_space=pltpu.SMEM),) * 5,
    )(partials.reshape(NW // 2, 2 * OUT_W), tcp, tcq,
      duration_target, duration_prediction, pitch_target, pitch_prediction,
      energy_target, energy_prediction, variance_mask, mask3d)
    return tuple(o.reshape(()) for o in outs)
